# Initial kernel scaffold; baseline (speedup 1.0000x reference)
import jax
import jax.numpy as jnp
from jax.experimental import pallas as pl


def kernel(cls_out_0, cls_out_1, cls_out_2, cls_out_3, cls_out_4, box_out_0, box_out_1, box_out_2, box_out_3, box_out_4, image_scales):
    raise NotImplementedError("write your pallas kernel here")


# speedup vs baseline: 1.3707x; 1.3707x over previous
import jax
import jax.numpy as jnp
import numpy as np
from jax.experimental import pallas as pl

_NUM_CLASSES = 90
_TOPK = 5000
_DETS = 100
_IOU_THR = 0.5
_ROWS, _LANES = 40, 128
_OUT_ROWS = 104


def _anchor_boxes_np():
    image_size = 512
    num_scales = 3
    aspect_ratios = [(1.0, 1.0), (1.4, 0.7), (0.7, 1.4)]
    anchor_scale = 4.0
    boxes_all = []
    for level in range(3, 8):
        stride = 2 ** level
        boxes_level = []
        for scale_octave in range(num_scales):
            for ar in aspect_ratios:
                base = anchor_scale * stride * 2 ** (scale_octave / float(num_scales))
                ax = base * ar[0] / 2.0
                ay = base * ar[1] / 2.0
                x = np.arange(stride / 2.0, image_size, stride)
                y = np.arange(stride / 2.0, image_size, stride)
                xv, yv = np.meshgrid(x, y)
                xv = xv.reshape(-1)
                yv = yv.reshape(-1)
                b = np.vstack((yv - ay, xv - ax, yv + ay, xv + ax)).T
                boxes_level.append(np.expand_dims(b, 1))
        boxes_all.append(np.concatenate(boxes_level, 1).reshape(-1, 4))
    return np.vstack(boxes_all).astype(np.float32)


def _nms_kernel(scale_ref, cls_ref, rel_ref, anc_ref, clsid_ref, o_ref):
    scale = scale_ref[0, 0, 0]
    ty = rel_ref[0, 0]
    tx = rel_ref[0, 1]
    th = rel_ref[0, 2]
    tw = rel_ref[0, 3]
    ay1 = anc_ref[0, 0]
    ax1 = anc_ref[0, 1]
    ay2 = anc_ref[0, 2]
    ax2 = anc_ref[0, 3]

    yca = (ay1 + ay2) * 0.5
    xca = (ax1 + ax2) * 0.5
    ha = ay2 - ay1
    wa = ax2 - ax1
    w = jnp.exp(tw) * wa
    h = jnp.exp(th) * ha
    yc = ty * ha + yca
    xc = tx * wa + xca
    y1 = yc - h * 0.5
    x1 = xc - w * 0.5
    y2 = yc + h * 0.5
    x2 = xc + w * 0.5
    areas = (y2 - y1) * (x2 - x1)

    row_i = jax.lax.broadcasted_iota(jnp.int32, (_ROWS, _LANES), 0)
    lane_i = jax.lax.broadcasted_iota(jnp.int32, (_ROWS, _LANES), 1)
    lin = row_i * _LANES + lane_i
    valid = lin < _TOPK
    neg_inf = jnp.float32(-jnp.inf)
    s0 = jnp.where(valid, jax.nn.sigmoid(cls_ref[0]), neg_inf)
    cls_f = clsid_ref[0]

    out_lane = jax.lax.broadcasted_iota(jnp.int32, (1, _LANES), 1)

    def body(i, s):
        m = jnp.max(s)
        mask = s == m
        jidx = jnp.min(jnp.where(mask, lin, _ROWS * _LANES))
        onehot = lin == jidx

        def sel(a):
            return jnp.sum(jnp.where(onehot, a, 0.0))

        by1 = sel(y1)
        bx1 = sel(x1)
        by2 = sel(y2)
        bx2 = sel(x2)
        aj = sel(areas)
        cj = sel(cls_f)

        iy1 = jnp.maximum(y1, by1)
        ix1 = jnp.maximum(x1, bx1)
        iy2 = jnp.minimum(y2, by2)
        ix2 = jnp.minimum(x2, bx2)
        inter = jnp.maximum(iy2 - iy1, 0.0) * jnp.maximum(ix2 - ix1, 0.0)
        iou = inter / (areas + aj - inter + 1e-8)
        s = jnp.where(iou > _IOU_THR, neg_inf, s)
        s = jnp.where(onehot, neg_inf, s)

        vals = (bx1 * scale, by1 * scale, (bx2 - bx1) * scale,
                (by2 - by1) * scale, m, cj + 1.0)
        row = jnp.zeros((1, _LANES), jnp.float32)
        for k, v in enumerate(vals):
            row = jnp.where(out_lane == k, v, row)
        o_ref[0, pl.ds(i, 1), :] = row
        return s

    jax.lax.fori_loop(0, _DETS, body, s0)


def kernel(cls_out_0, cls_out_1, cls_out_2, cls_out_3, cls_out_4,
           box_out_0, box_out_1, box_out_2, box_out_3, box_out_4,
           image_scales):
    cls_list = [cls_out_0, cls_out_1, cls_out_2, cls_out_3, cls_out_4]
    box_list = [box_out_0, box_out_1, box_out_2, box_out_3, box_out_4]
    b = cls_list[0].shape[0]
    cls_all = jnp.concatenate(
        [jnp.transpose(c, (0, 2, 3, 1)).reshape(b, -1, _NUM_CLASSES)
         for c in cls_list], axis=1)
    box_all = jnp.concatenate(
        [jnp.transpose(x, (0, 2, 3, 1)).reshape(b, -1, 4)
         for x in box_list], axis=1)

    flat = cls_all.reshape(b, -1)
    _, topk_idx = jax.lax.top_k(flat, _TOPK)
    indices = topk_idx // _NUM_CLASSES
    classes = topk_idx % _NUM_CLASSES
    box_topk = jnp.take_along_axis(box_all, indices[:, :, None], axis=1)
    cls_at = jnp.take_along_axis(cls_all, indices[:, :, None], axis=1)
    cls_topk = jnp.take_along_axis(cls_at, classes[:, :, None], axis=2)[..., 0]

    anchors = jnp.asarray(_anchor_boxes_np())
    anchors_sel = anchors[indices]

    pad = _ROWS * _LANES - _TOPK

    def pad_planes(x):
        x = jnp.transpose(x, (0, 2, 1))
        x = jnp.pad(x, ((0, 0), (0, 0), (0, pad)))
        return x.reshape(b, 4, _ROWS, _LANES)

    rel_p = pad_planes(box_topk)
    anc_p = pad_planes(anchors_sel)
    cls_p = jnp.pad(cls_topk, ((0, 0), (0, pad))).reshape(b, _ROWS, _LANES)
    clsid_p = jnp.pad(classes.astype(jnp.float32),
                      ((0, 0), (0, pad))).reshape(b, _ROWS, _LANES)
    scales_p = jnp.broadcast_to(image_scales.reshape(b, 1, 1), (b, 8, 128))

    out = pl.pallas_call(
        _nms_kernel,
        grid=(b,),
        in_specs=[
            pl.BlockSpec((1, 8, 128), lambda i: (i, 0, 0)),
            pl.BlockSpec((1, _ROWS, _LANES), lambda i: (i, 0, 0)),
            pl.BlockSpec((1, 4, _ROWS, _LANES), lambda i: (i, 0, 0, 0)),
            pl.BlockSpec((1, 4, _ROWS, _LANES), lambda i: (i, 0, 0, 0)),
            pl.BlockSpec((1, _ROWS, _LANES), lambda i: (i, 0, 0)),
        ],
        out_specs=pl.BlockSpec((1, _OUT_ROWS, _LANES), lambda i: (i, 0, 0)),
        out_shape=jax.ShapeDtypeStruct((b, _OUT_ROWS, _LANES), jnp.float32),
    )(scales_p, cls_p, rel_p, anc_p, clsid_p)

    return out[:, :_DETS, :6]
